# baseline (speedup 1.0000x reference)
import functools

import jax
import jax.numpy as jnp
from jax import lax
from jax.experimental import pallas as pl
from jax.experimental.pallas import tpu as pltpu
from jax.experimental.pallas import tpu_sc as plsc

NUM_CATEGORIES = 1000000
L = 26
D = 16
B = 16384
N = B * L

NC = 2
NS = 16
NW = NC * NS
PER_W = N // NW

IDX_ROW = 128
IDX_ROWS_W = PER_W // IDX_ROW
CHUNK = 1664
DMAS_PER_CHUNK = CHUNK // IDX_ROW
CHUNKS = PER_W // CHUNK
GROUPS = CHUNK // L


def _body(x_hbm, table_hbm, pw_hbm, pb_hbm, out_hbm,
          idx_v, buf_v, pw_v, pb_v, gsem):
    wid = lax.axis_index("s") * NC + lax.axis_index("c")
    base_idx_row = wid * IDX_ROWS_W
    base_out = wid * PER_W

    pltpu.sync_copy(x_hbm.at[pl.ds(base_idx_row, IDX_ROWS_W)], idx_v)
    pltpu.sync_copy(pw_hbm, pw_v)
    pltpu.sync_copy(pb_hbm, pb_v)

    def chunk_body(c, carry):
        copies = []
        for j in range(DMAS_PER_CHUNK):
            copies.append(pltpu.async_copy(
                table_hbm.at[idx_v.at[c * DMAS_PER_CHUNK + j]],
                buf_v.at[pl.ds(j * IDX_ROW, IDX_ROW)],
                gsem))
        for cp in copies:
            cp.wait()

        def group_body(g, carry2):
            row0 = g * L
            for l in range(L):
                r = row0 + l
                buf_v[r] = buf_v[r] * pw_v[l] + pb_v[l]
            return carry2
        lax.fori_loop(0, GROUPS, group_body, 0)

        pltpu.sync_copy(buf_v, out_hbm.at[pl.ds(base_out + c * CHUNK, CHUNK)])
        return carry

    lax.fori_loop(0, CHUNKS, chunk_body, 0)


@jax.jit
def kernel(x, shared_embed, position_weights, position_bias):
    x_flat = x.reshape(N // IDX_ROW, IDX_ROW)
    mesh = plsc.VectorSubcoreMesh(core_axis_name="c", subcore_axis_name="s")
    out_flat = pl.kernel(
        _body,
        out_type=jax.ShapeDtypeStruct((N, D), jnp.float32),
        mesh=mesh,
        compiler_params=pltpu.CompilerParams(use_tc_tiling_on_sc=False),
        scratch_types=[
            pltpu.VMEM((IDX_ROWS_W, IDX_ROW), jnp.int32),
            pltpu.VMEM((CHUNK, D), jnp.float32),
            pltpu.VMEM((L, D), jnp.float32),
            pltpu.VMEM((L, D), jnp.float32),
            pltpu.SemaphoreType.DMA,
        ],
    )(x_flat, shared_embed, position_weights, position_bias)
    return out_flat.reshape(B, L, D)


# speedup vs baseline: 1.2755x; 1.2755x over previous
import jax
import jax.numpy as jnp
from jax import lax
from jax.experimental import pallas as pl
from jax.experimental.pallas import tpu as pltpu
from jax.experimental.pallas import tpu_sc as plsc

NUM_CATEGORIES = 1000000
L = 26
D = 16
B = 16384

NC = 2
NS = 16
NW = NC * NS
B_W = B // NW

CB = 64
CHUNKS = B_W // CB
GSTEP = 4
FIRES = CB // GSTEP


def _body(x_hbm, table_hbm, pw_hbm, pb_hbm, out_hbm,
          idx_v, buf_v, pw_v, pb_v, gsem):
    wid = lax.axis_index("s") * NC + lax.axis_index("c")
    b0 = wid * B_W

    pltpu.sync_copy(x_hbm.at[pl.ds(b0, B_W)], idx_v)
    pltpu.sync_copy(pw_hbm, pw_v)
    pltpu.sync_copy(pb_hbm, pb_v)

    def fire(c, p):
        def fire_one(g, carry):
            pltpu.async_copy(
                table_hbm.at[idx_v.at[c * CB + g]],
                buf_v.at[p, g],
                gsem.at[p])
            return carry
        lax.fori_loop(0, CB, fire_one, 0)

    def drain(c, p):
        pltpu.make_async_copy(
            out_hbm.at[pl.ds(b0 + c * CB, CB)], buf_v.at[p],
            gsem.at[p]).wait()

    fire(0, 0)

    def chunk_body(c, carry):
        p = lax.rem(c, 2)

        @pl.when(c + 1 < CHUNKS)
        def _():
            fire(c + 1, 1 - p)

        drain(c, p)

        def group_body(g, carry2):
            for l in range(L):
                buf_v[p, g, l] = buf_v[p, g, l] * pw_v[l] + pb_v[l]
            return carry2
        lax.fori_loop(0, CB, group_body, 0)

        pltpu.sync_copy(buf_v.at[p], out_hbm.at[pl.ds(b0 + c * CB, CB)])
        return carry

    lax.fori_loop(0, CHUNKS, chunk_body, 0)


@jax.jit
def kernel(x, shared_embed, position_weights, position_bias):
    mesh = plsc.VectorSubcoreMesh(core_axis_name="c", subcore_axis_name="s")
    return pl.kernel(
        _body,
        out_type=jax.ShapeDtypeStruct((B, L, D), jnp.float32),
        mesh=mesh,
        compiler_params=pltpu.CompilerParams(use_tc_tiling_on_sc=False),
        scratch_types=[
            pltpu.VMEM((B_W, L), jnp.int32),
            pltpu.VMEM((2, CB, L, D), jnp.float32),
            pltpu.VMEM((L, D), jnp.float32),
            pltpu.VMEM((L, D), jnp.float32),
            pltpu.SemaphoreType.DMA((2,)),
        ],
    )(x, shared_embed, position_weights, position_bias)
